# baseline (speedup 1.0000x reference)
import jax
import jax.numpy as jnp
from jax.experimental import pallas as pl


def _add_kernel(x_ref, t_ref, o_ref):
    o_ref[...] = x_ref[...] + t_ref[...]


def kernel(x, table):
    B, S, H = x.shape
    grid = (B,)
    return pl.pallas_call(
        _add_kernel,
        grid=grid,
        in_specs=[
            pl.BlockSpec((1, S, H), lambda b: (b, 0, 0)),
            pl.BlockSpec((S, H), lambda b: (0, 0)),
        ],
        out_specs=pl.BlockSpec((1, S, H), lambda b: (b, 0, 0)),
        out_shape=jax.ShapeDtypeStruct((B, S, H), x.dtype),
    )(x, table[:S])
